# baseline (speedup 1.0000x reference)
import jax
import jax.numpy as jnp
from jax import lax
from jax.experimental import pallas as pl
from jax.experimental.pallas import tpu as pltpu
from jax.experimental.pallas import tpu_sc as plsc

BATCH = 1024
SEQ = 200
DIM = 64
NC = 2
NS = 16
NW = NC * NS
SEQ_PER_W = BATCH // NW
HALF = SEQ // 2
LANES = 16


def _emb_body(idx_hbm, table_hbm, pos_hbm, out_hbm, pos_v, idx_v, rows_v, gsem):
    cid = lax.axis_index("c")
    sid = lax.axis_index("s")
    wid = sid * NC + cid

    pltpu.sync_copy(pos_hbm, pos_v)

    def seq_body(t, carry):
        b = wid * SEQ_PER_W + t
        pltpu.sync_copy(idx_hbm.at[b], idx_v)
        cp0 = pltpu.async_copy(
            table_hbm.at[idx_v.at[0]], rows_v.at[pl.ds(0, HALF)], gsem)
        cp1 = pltpu.async_copy(
            table_hbm.at[idx_v.at[1]], rows_v.at[pl.ds(HALF, HALF)], gsem)
        cp0.wait()
        cp1.wait()

        def row_body(r, c2):
            for j in range(DIM // LANES):
                s = pl.ds(j * LANES, LANES)
                rows_v[r, s] = rows_v[r, s] + pos_v[r, s]
            return c2

        lax.fori_loop(0, SEQ, row_body, 0)
        pltpu.sync_copy(rows_v, out_hbm.at[b])
        return carry

    lax.fori_loop(0, SEQ_PER_W, seq_body, 0)


@jax.jit
def kernel(inputs, word_table, pos_table):
    idx = inputs.astype(jnp.int32).reshape(BATCH, 2, HALF)
    mesh = plsc.VectorSubcoreMesh(core_axis_name="c", subcore_axis_name="s")
    return pl.kernel(
        _emb_body,
        mesh=mesh,
        out_type=jax.ShapeDtypeStruct((BATCH, SEQ, DIM), jnp.float32),
        scratch_types=[
            pltpu.VMEM((SEQ, DIM), jnp.float32),
            pltpu.VMEM((2, HALF), jnp.int32),
            pltpu.VMEM((SEQ, DIM), jnp.float32),
            pltpu.SemaphoreType.DMA,
        ],
        compiler_params=pltpu.CompilerParams(use_tc_tiling_on_sc=False),
    )(idx, word_table, pos_table)


# speedup vs baseline: 1.0658x; 1.0658x over previous
import jax
import jax.numpy as jnp
from jax import lax
from jax.experimental import pallas as pl
from jax.experimental.pallas import tpu as pltpu
from jax.experimental.pallas import tpu_sc as plsc

BATCH = 1024
SEQ = 200
DIM = 64
NC = 2
NS = 16
NW = NC * NS
SEQ_PER_W = BATCH // NW
HALF = SEQ // 2
LANES = 16


def _emb_body(idx_hbm, table_hbm, pos_hbm, out_hbm,
              pos_v, idx_all, rows0, rows1, gsem0, gsem1, osem0, osem1):
    cid = lax.axis_index("c")
    sid = lax.axis_index("s")
    wid = sid * NC + cid
    base = wid * SEQ_PER_W

    pltpu.sync_copy(idx_hbm.at[wid], idx_all)
    pltpu.sync_copy(pos_hbm, pos_v)

    def issue_gathers(t, rows, sem):
        pltpu.async_copy(table_hbm.at[idx_all.at[t, 0]],
                         rows.at[pl.ds(0, HALF)], sem)
        pltpu.async_copy(table_hbm.at[idx_all.at[t, 1]],
                         rows.at[pl.ds(HALF, HALF)], sem)

    def wait_gathers(rows, sem):
        pltpu.make_async_copy(table_hbm.at[idx_all.at[0, 0]],
                              rows.at[pl.ds(0, HALF)], sem).wait()
        pltpu.make_async_copy(table_hbm.at[idx_all.at[0, 1]],
                              rows.at[pl.ds(HALF, HALF)], sem).wait()

    def wait_out(rows, sem):
        pltpu.make_async_copy(rows, out_hbm.at[base], sem).wait()

    def add_and_out(t, rows, osem):
        @plsc.parallel_loop(0, SEQ, 1, unroll=4)
        def _add(r):
            for j in range(DIM // LANES):
                s = pl.ds(j * LANES, LANES)
                rows[r, s] = rows[r, s] + pos_v[r, s]

        pltpu.async_copy(rows, out_hbm.at[base + t], osem)

    issue_gathers(0, rows0, gsem0)

    def pair_body(p, carry):
        t0 = 2 * p
        wait_gathers(rows0, gsem0)

        @pl.when(p >= 1)
        def _():
            wait_out(rows1, osem1)

        issue_gathers(t0 + 1, rows1, gsem1)
        add_and_out(t0, rows0, osem0)

        wait_gathers(rows1, gsem1)
        wait_out(rows0, osem0)

        @pl.when(p < SEQ_PER_W // 2 - 1)
        def _():
            issue_gathers(t0 + 2, rows0, gsem0)

        add_and_out(t0 + 1, rows1, osem1)
        return carry

    lax.fori_loop(0, SEQ_PER_W // 2, pair_body, 0)
    wait_out(rows1, osem1)


@jax.jit
def kernel(inputs, word_table, pos_table):
    idx = inputs.astype(jnp.int32).reshape(NW, SEQ_PER_W, 2, HALF)
    mesh = plsc.VectorSubcoreMesh(core_axis_name="c", subcore_axis_name="s")
    return pl.kernel(
        _emb_body,
        mesh=mesh,
        out_type=jax.ShapeDtypeStruct((BATCH, SEQ, DIM), jnp.float32),
        scratch_types=[
            pltpu.VMEM((SEQ, DIM), jnp.float32),
            pltpu.VMEM((SEQ_PER_W, 2, HALF), jnp.int32),
            pltpu.VMEM((SEQ, DIM), jnp.float32),
            pltpu.VMEM((SEQ, DIM), jnp.float32),
            pltpu.SemaphoreType.DMA,
            pltpu.SemaphoreType.DMA,
            pltpu.SemaphoreType.DMA,
            pltpu.SemaphoreType.DMA,
        ],
        compiler_params=pltpu.CompilerParams(use_tc_tiling_on_sc=False),
    )(idx, word_table, pos_table)


# speedup vs baseline: 1.0660x; 1.0002x over previous
import jax
import jax.numpy as jnp
from jax import lax
from jax.experimental import pallas as pl
from jax.experimental.pallas import tpu as pltpu
from jax.experimental.pallas import tpu_sc as plsc

BATCH = 1024
SEQ = 200
DIM = 64
NC = 2
NS = 16
NW = NC * NS
SEQ_PER_W = BATCH // NW
SPLIT0 = 96
SPLIT1 = SEQ - SPLIT0
LANES = 16


def _emb_body(idx_hbm, table_hbm, pos_hbm, out_hbm,
              pos_v, idx_all, rows0, rows1, gsem0, gsem1, osem0, osem1):
    cid = lax.axis_index("c")
    sid = lax.axis_index("s")
    wid = sid * NC + cid
    base = wid * SEQ_PER_W

    pltpu.sync_copy(idx_hbm.at[pl.ds(base, SEQ_PER_W)], idx_all)
    pltpu.sync_copy(pos_hbm, pos_v)

    def issue_gathers(t, rows, sem):
        pltpu.async_copy(table_hbm.at[idx_all.at[t, pl.ds(0, SPLIT0)]],
                         rows.at[pl.ds(0, SPLIT0)], sem)
        pltpu.async_copy(table_hbm.at[idx_all.at[t, pl.ds(SPLIT0, SPLIT1)]],
                         rows.at[pl.ds(SPLIT0, SPLIT1)], sem)

    def wait_gathers(rows, sem):
        pltpu.make_async_copy(table_hbm.at[idx_all.at[0, pl.ds(0, SPLIT0)]],
                              rows.at[pl.ds(0, SPLIT0)], sem).wait()
        pltpu.make_async_copy(table_hbm.at[idx_all.at[0, pl.ds(SPLIT0, SPLIT1)]],
                              rows.at[pl.ds(SPLIT0, SPLIT1)], sem).wait()

    def wait_out(rows, sem):
        pltpu.make_async_copy(rows, out_hbm.at[base], sem).wait()

    def add_and_out(t, rows, osem):
        @plsc.parallel_loop(0, SEQ, 1, unroll=4)
        def _add(r):
            for j in range(DIM // LANES):
                s = pl.ds(j * LANES, LANES)
                rows[r, s] = rows[r, s] + pos_v[r, s]

        pltpu.async_copy(rows, out_hbm.at[base + t], osem)

    issue_gathers(0, rows0, gsem0)

    def pair_body(p, carry):
        t0 = 2 * p
        wait_gathers(rows0, gsem0)

        @pl.when(p >= 1)
        def _():
            wait_out(rows1, osem1)

        issue_gathers(t0 + 1, rows1, gsem1)
        add_and_out(t0, rows0, osem0)

        wait_gathers(rows1, gsem1)
        wait_out(rows0, osem0)

        @pl.when(p < SEQ_PER_W // 2 - 1)
        def _():
            issue_gathers(t0 + 2, rows0, gsem0)

        add_and_out(t0 + 1, rows1, osem1)
        return carry

    lax.fori_loop(0, SEQ_PER_W // 2, pair_body, 0)
    wait_out(rows1, osem1)


@jax.jit
def kernel(inputs, word_table, pos_table):
    idx = inputs.astype(jnp.int32)
    mesh = plsc.VectorSubcoreMesh(core_axis_name="c", subcore_axis_name="s")
    return pl.kernel(
        _emb_body,
        mesh=mesh,
        out_type=jax.ShapeDtypeStruct((BATCH, SEQ, DIM), jnp.float32),
        scratch_types=[
            pltpu.VMEM((SEQ, DIM), jnp.float32),
            pltpu.VMEM((SEQ_PER_W, SEQ), jnp.int32),
            pltpu.VMEM((SEQ, DIM), jnp.float32),
            pltpu.VMEM((SEQ, DIM), jnp.float32),
            pltpu.SemaphoreType.DMA,
            pltpu.SemaphoreType.DMA,
            pltpu.SemaphoreType.DMA,
            pltpu.SemaphoreType.DMA,
        ],
        compiler_params=pltpu.CompilerParams(use_tc_tiling_on_sc=False),
    )(idx, word_table, pos_table)
